# baseline (speedup 1.0000x reference)
import functools

import jax
import jax.numpy as jnp
from jax import lax
from jax.experimental import pallas as pl
from jax.experimental.pallas import tpu as pltpu
from jax.experimental.pallas import tpu_sc as plsc

BATCH = 4096
HIST = 200
DIM = 32
LANES = 128

_info = plsc.get_sparse_core_info()
NC, NS = _info.num_cores, _info.num_subcores
NW = NC * NS
TOTAL = BATCH * HIST
PER_W = TOTAL // NW
CHUNK = 128
G = PER_W // CHUNK

VROWS = 1000001
VPAD = 1000008
BLK = 128
NFULL = (VROWS + BLK - 1) // BLK
LAST = VROWS - (NFULL - 1) * BLK
RBUF = 4
KPT = (NFULL + NS - 1) // NS
ROUNDS = (KPT + RBUF - 1) // RBUF


def _repack(src, dst, rows):
    def body(r0, carry):
        for dr in range(8):
            r = r0 + dr
            for c in range(0, DIM, 16):
                dst[r, pl.ds(c, 16)] = src[r, pl.ds(c, 16)]
        return carry
    lax.fori_loop(0, rows // 8, lambda i, c: body(i * 8, c), 0, unroll=2)


def _make_kernel(vocab_rows):
    mesh = plsc.VectorSubcoreMesh(core_axis_name="c", subcore_axis_name="s")

    @functools.partial(
        pl.kernel,
        out_type=jax.ShapeDtypeStruct((TOTAL, DIM), jnp.float32),
        mesh=mesh,
        scratch_types=[
            pltpu.HBM((VPAD, LANES), jnp.float32),
            pltpu.VMEM((G, CHUNK), jnp.int32),
            pltpu.VMEM((BLK, DIM), jnp.float32),
            pltpu.VMEM((BLK, DIM), jnp.float32),
        ] + [pltpu.VMEM((BLK, LANES), jnp.float32) for _ in range(RBUF)] + [
            pltpu.SemaphoreType.DMA,
            pltpu.SemaphoreType.DMA,
            pltpu.SemaphoreType.DMA,
        ],
    )
    def k(idx_hbm, table_hbm, out_hbm, scratch_hbm, idx_v, s0, s1,
          b0, b1, b2, b3, wsem, gsem, osem):
        sid = lax.axis_index("s")
        cid = lax.axis_index("c")
        wid = sid * NC + cid
        bufs = (b0, b1, b2, b3)
        stages = (s0, s1)

        def relay_round(j, carry):
            for b in range(RBUF):
                kblk = (j * RBUF + b) * NS + sid
                kprev = ((j - 1) * RBUF + b) * NS + sid
                n = kblk * BLK

                @pl.when((j > 0) & (kprev < NFULL - 1))
                def _():
                    pltpu.make_async_copy(
                        bufs[b].at[:, pl.ds(0, DIM)],
                        scratch_hbm.at[pl.ds(0, BLK), pl.ds(0, DIM)],
                        wsem).wait()

                @pl.when(kblk < NFULL - 1)
                def _():
                    pltpu.sync_copy(table_hbm.at[pl.ds(n, BLK)], s0)
                    _repack(s0, bufs[b], BLK)
                    pltpu.async_copy(
                        bufs[b].at[:, pl.ds(0, DIM)],
                        scratch_hbm.at[pl.ds(n, BLK), pl.ds(0, DIM)], wsem)
            return carry

        lax.fori_loop(0, ROUNDS, relay_round, 0)

        for b in range(RBUF):
            kblk = ((ROUNDS - 1) * RBUF + b) * NS + sid
            @pl.when(kblk < NFULL - 1)
            def _():
                pltpu.make_async_copy(
                    bufs[b].at[:, pl.ds(0, DIM)],
                    scratch_hbm.at[pl.ds(0, BLK), pl.ds(0, DIM)], wsem).wait()

        @pl.when(sid == NS - 1)
        def _():
            n = (NFULL - 1) * BLK
            pltpu.sync_copy(
                table_hbm.at[pl.ds(n, LAST)], s1.at[pl.ds(0, LAST)])
            _repack(s1, b1, 64)
            for r in range(64, LAST):
                for c in range(0, DIM, 16):
                    b1[r, pl.ds(c, 16)] = s1[r, pl.ds(c, 16)]
            pltpu.async_copy(
                b1.at[pl.ds(0, LAST), pl.ds(0, DIM)],
                scratch_hbm.at[pl.ds(n, LAST), pl.ds(0, DIM)], wsem)
            pltpu.make_async_copy(
                b1.at[pl.ds(0, LAST), pl.ds(0, DIM)],
                scratch_hbm.at[pl.ds(0, LAST), pl.ds(0, DIM)], wsem).wait()
        plsc.subcore_barrier()

        pltpu.sync_copy(idx_hbm.at[wid], idx_v)
        obase = wid * PER_W

        def fire(g, b):
            pltpu.async_copy(scratch_hbm.at[idx_v.at[g]], bufs[b], gsem)

        def drain(g, b):
            pltpu.make_async_copy(
                scratch_hbm.at[idx_v.at[g]], bufs[b], gsem).wait()

        def wait_store(g, s):
            pltpu.make_async_copy(
                stages[s],
                out_hbm.at[pl.ds(obase + g * CHUNK, CHUNK)], osem).wait()

        def store(g, s):
            pltpu.async_copy(
                stages[s],
                out_hbm.at[pl.ds(obase + g * CHUNK, CHUNK)], osem)

        for b in range(RBUF):
            fire(b, b)

        def gather_round(j, carry):
            for b in range(RBUF):
                g = j * RBUF + b
                s = b % 2
                drain(g, b)
                @pl.when(g >= 2)
                def _():
                    wait_store(g - 2, s)
                _repack(bufs[b], stages[s], BLK)
                store(g, s)
                fire(g + RBUF, b)
            return carry

        lax.fori_loop(0, G // RBUF - 1, gather_round, 0)
        for b in range(RBUF):
            g = G - RBUF + b
            s = b % 2
            drain(g, b)
            wait_store(g - 2, s)
            _repack(bufs[b], stages[s], BLK)
            store(g, s)
        wait_store(G - 2, 0)
        wait_store(G - 1, 1)

    return k


def kernel(indices, table):
    idx = indices.astype(jnp.int32).reshape(NW, G, CHUNK)
    out = _make_kernel(table.shape[0])(idx, table)
    return out.reshape(BATCH, HIST, DIM)


# speedup vs baseline: 2.2018x; 2.2018x over previous
import functools

import jax
import jax.numpy as jnp
from jax import lax
from jax.experimental import pallas as pl
from jax.experimental.pallas import tpu as pltpu
from jax.experimental.pallas import tpu_sc as plsc

BATCH = 4096
HIST = 200
DIM = 32

_info = plsc.get_sparse_core_info()
NC, NS = _info.num_cores, _info.num_subcores
NW = NC * NS
TOTAL = BATCH * HIST
PER_W = TOTAL // NW
CHUNK = 128
G = PER_W // CHUNK


SUP = 10
NSUP = G // SUP
ROWS = SUP * CHUNK


def _make_kernel(vocab_rows):
    mesh = plsc.VectorSubcoreMesh(core_axis_name="c", subcore_axis_name="s")

    @functools.partial(
        pl.kernel,
        out_type=jax.ShapeDtypeStruct((NW, NSUP, ROWS, DIM), jnp.float32),
        mesh=mesh,
        scratch_types=[
            pltpu.VMEM((G, CHUNK), jnp.int32),
            pltpu.VMEM((ROWS, DIM), jnp.float32),
            pltpu.VMEM((ROWS, DIM), jnp.float32),
            pltpu.SemaphoreType.DMA,
        ],
        compiler_params=pltpu.CompilerParams(use_tc_tiling_on_sc=False),
    )
    def gather_kernel(idx_hbm, table_hbm, out_hbm, idx_v, buf0, buf1, gsem):
        wid = lax.axis_index("s") * NC + lax.axis_index("c")
        pltpu.sync_copy(idx_hbm.at[wid], idx_v)

        def fire(s, buf):
            for k in range(SUP):
                pltpu.async_copy(
                    table_hbm.at[idx_v.at[s * SUP + k]],
                    buf.at[pl.ds(k * CHUNK, CHUNK)], gsem)

        def drain(s, buf):
            for k in range(SUP):
                pltpu.make_async_copy(
                    table_hbm.at[idx_v.at[s * SUP + k]],
                    buf.at[pl.ds(k * CHUNK, CHUNK)], gsem).wait()

        fire(0, buf0)
        fire(1, buf1)

        def body(i, carry):
            s0 = 2 * i
            drain(s0, buf0)
            pltpu.sync_copy(buf0, out_hbm.at[wid, s0])
            fire(s0 + 2, buf0)
            drain(s0 + 1, buf1)
            pltpu.sync_copy(buf1, out_hbm.at[wid, s0 + 1])
            fire(s0 + 3, buf1)
            return carry

        lax.fori_loop(0, NSUP // 2 - 1, body, 0)

        drain(NSUP - 2, buf0)
        pltpu.sync_copy(buf0, out_hbm.at[wid, NSUP - 2])
        drain(NSUP - 1, buf1)
        pltpu.sync_copy(buf1, out_hbm.at[wid, NSUP - 1])

    return gather_kernel


def kernel(indices, table):
    idx = indices.astype(jnp.int32).reshape(NW, G, CHUNK)
    out = _make_kernel(table.shape[0])(idx, table)
    return out.reshape(BATCH, HIST, DIM)
